# baseline (speedup 1.0000x reference)
import functools

import jax
import jax.numpy as jnp
from jax import lax
from jax.experimental import pallas as pl
from jax.experimental.pallas import tpu as pltpu
from jax.experimental.pallas import tpu_sc as plsc

_B = 4
_T = 4096
_D = 768
_N = _B * _T
_NC = 2
_NS = 16
_NW = _NC * _NS
_P_W = _T // _NW
_CK = 16
_KPB = _P_W // _CK
_NCH = _B * _KPB
_LANES = 16
_GRP = _D // _LANES


def _make_emb_kernel():
    mesh = plsc.VectorSubcoreMesh(core_axis_name="c", subcore_axis_name="s")

    @functools.partial(
        pl.kernel,
        out_type=jax.ShapeDtypeStruct((_N, _D), jnp.float32),
        mesh=mesh,
        scratch_types=[
            pltpu.VMEM((_NCH, _CK), jnp.int32),
            pltpu.VMEM((_P_W, _D), jnp.float32),
            pltpu.VMEM((_CK, _D), jnp.float32),
            pltpu.VMEM((_CK, _D), jnp.float32),
            pltpu.SemaphoreType.DMA,
            pltpu.SemaphoreType.DMA,
            pltpu.SemaphoreType.DMA,
        ],
    )
    def emb(xr_hbm, tok_hbm, pos_hbm, out_hbm,
            idx_v, pos_v, tok0_v, tok1_v, psem, gsem, ssem):
        wid = lax.axis_index("s") * _NC + lax.axis_index("c")
        p0 = pl.multiple_of(wid * _P_W, _P_W)
        bufs = (tok0_v, tok1_v)

        pos_cp = pltpu.async_copy(pos_hbm.at[pl.ds(p0, _P_W)], pos_v, psem)
        for b in range(_B):
            r0 = b * _T + p0
            pltpu.sync_copy(xr_hbm.at[pl.ds(pl.multiple_of(r0 // _CK, _KPB),
                                            _KPB)],
                            idx_v.at[pl.ds(b * _KPB, _KPB)])

        def out_rows(j):
            return pl.multiple_of(
                (j // _KPB) * _T + p0 + lax.rem(j, _KPB) * _CK, _CK)

        pltpu.async_copy(tok_hbm.at[idx_v.at[0]], tok0_v, gsem)
        pos_cp.wait()

        def step(si, _):
            for rb in range(2):
                j = si * 2 + rb
                buf = bufs[rb]
                nbuf = bufs[rb ^ 1]
                pltpu.make_async_copy(tok_hbm.at[idx_v.at[0]], buf, gsem).wait()
                @pl.when(j > 0)
                def _():
                    pltpu.make_async_copy(nbuf, out_hbm.at[pl.ds(0, _CK)],
                                          ssem).wait()
                @pl.when(j < _NCH - 1)
                def _():
                    pltpu.async_copy(tok_hbm.at[idx_v.at[j + 1]], nbuf, gsem)
                pb = lax.rem(j, _KPB) * _CK

                def row_body(i, _):
                    for g in range(_GRP):
                        s = pl.ds(g * _LANES, _LANES)
                        buf[i, s] = buf[i, s] + pos_v[pb + i, s]
                    return 0

                lax.fori_loop(0, _CK, row_body, 0)
                pltpu.async_copy(buf, out_hbm.at[pl.ds(out_rows(j), _CK)], ssem)
            return 0

        lax.fori_loop(0, _NCH // 2, step, 0)
        pltpu.make_async_copy(tok1_v, out_hbm.at[pl.ds(0, _CK)], ssem).wait()

    return emb


_emb = _make_emb_kernel()


def kernel(x, token_table, pos_table):
    xr = x.reshape(_N // _CK, _CK).astype(jnp.int32)
    out = _emb(xr, token_table, pos_table)
    return out.reshape(_B, _T, _D)


# speedup vs baseline: 1.0495x; 1.0495x over previous
import functools

import jax
import jax.numpy as jnp
from jax import lax
from jax.experimental import pallas as pl
from jax.experimental.pallas import tpu as pltpu
from jax.experimental.pallas import tpu_sc as plsc

_B = 4
_T = 4096
_D = 768
_N = _B * _T
_NC = 2
_NS = 16
_NW = _NC * _NS
_P_W = _T // _NW
_CK = 16
_KPB = _P_W // _CK
_NCH = _B * _KPB
_LANES = 16
_GRP = _D // _LANES


def _make_emb_kernel():
    mesh = plsc.VectorSubcoreMesh(core_axis_name="c", subcore_axis_name="s")

    @functools.partial(
        pl.kernel,
        out_type=jax.ShapeDtypeStruct((_N, _D), jnp.float32),
        mesh=mesh,
        scratch_types=[
            pltpu.VMEM((_NCH, _CK), jnp.int32),
            pltpu.VMEM((_P_W, _D), jnp.float32),
            pltpu.VMEM((_CK, _D), jnp.float32),
            pltpu.VMEM((_CK, _D), jnp.float32),
            pltpu.SemaphoreType.DMA,
            pltpu.SemaphoreType.DMA,
            pltpu.SemaphoreType.DMA,
        ],
    )
    def emb(xr_hbm, tok_hbm, pos_hbm, out_hbm,
            idx_v, pos_v, tok0_v, tok1_v, psem, gsem, ssem):
        wid = lax.axis_index("s") * _NC + lax.axis_index("c")
        p0 = pl.multiple_of(wid * _P_W, _P_W)
        bufs = (tok0_v, tok1_v)

        pos_cp = pltpu.async_copy(pos_hbm.at[pl.ds(p0, _P_W)], pos_v, psem)
        for b in range(_B):
            r0 = b * _T + p0
            pltpu.sync_copy(xr_hbm.at[pl.ds(pl.multiple_of(r0 // _CK, _KPB),
                                            _KPB)],
                            idx_v.at[pl.ds(b * _KPB, _KPB)])

        def gather(c):
            return pltpu.async_copy(tok_hbm.at[idx_v.at[c]],
                                    bufs[c % 2], gsem)

        def store(c):
            rows = pl.multiple_of(
                (c // _KPB) * _T + p0 + (c % _KPB) * _CK, _CK)
            return pltpu.async_copy(bufs[c % 2], out_hbm.at[pl.ds(rows, _CK)],
                                    ssem)

        g_cp = {0: gather(0)}
        s_cp = {}
        pos_cp.wait()
        for c in range(_NCH):
            buf = bufs[c % 2]
            g_cp.pop(c).wait()
            if c - 1 in s_cp:
                s_cp.pop(c - 1).wait()
            if c + 1 < _NCH:
                g_cp[c + 1] = gather(c + 1)
            pb = (c % _KPB) * _CK

            def row_body(i, _, pb=pb, buf=buf):
                for g in range(_GRP):
                    s = pl.ds(g * _LANES, _LANES)
                    buf[i, s] = buf[i, s] + pos_v[pb + i, s]
                return 0

            lax.fori_loop(0, _CK, row_body, 0)
            s_cp[c] = store(c)
        s_cp.pop(_NCH - 1).wait()

    return emb


_emb = _make_emb_kernel()


def kernel(x, token_table, pos_table):
    xr = x.reshape(_N // _CK, _CK).astype(jnp.int32)
    out = _emb(xr, token_table, pos_table)
    return out.reshape(_B, _T, _D)


# speedup vs baseline: 1.2942x; 1.2331x over previous
import functools

import jax
import jax.numpy as jnp
from jax import lax
from jax.experimental import pallas as pl
from jax.experimental.pallas import tpu as pltpu
from jax.experimental.pallas import tpu_sc as plsc

_B = 4
_T = 4096
_D = 768
_N = _B * _T
_NC = 2
_NS = 16
_NW = _NC * _NS
_P_W = _T // _NW
_CK = 32
_KPB = _P_W // _CK
_NCH = _B * _KPB // 2
_HALF = _P_W // 2
_LANES = 16
_GRP = _D // _LANES

_SCHED = [(h, b, k) for h in (0, 1) for b in range(_B)
          for k in (2 * h, 2 * h + 1)]


def _make_emb_kernel():
    mesh = plsc.VectorSubcoreMesh(core_axis_name="c", subcore_axis_name="s")

    @functools.partial(
        pl.kernel,
        out_type=jax.ShapeDtypeStruct((_N, _D), jnp.float32),
        mesh=mesh,
        scratch_types=[
            pltpu.VMEM((_B * _KPB, _CK), jnp.int32),
            pltpu.VMEM((_HALF, _D), jnp.float32),
            pltpu.VMEM((_CK, _D), jnp.float32),
            pltpu.VMEM((_CK, _D), jnp.float32),
            pltpu.SemaphoreType.DMA,
            pltpu.SemaphoreType.DMA,
            pltpu.SemaphoreType.DMA,
        ],
    )
    def emb(xw_hbm, tok_hbm, pos_hbm, out_hbm,
            idx_v, pos_v, tok0_v, tok1_v, psem, gsem, ssem):
        wid = lax.axis_index("s") * _NC + lax.axis_index("c")
        p0 = pl.multiple_of(wid * _P_W, _P_W)
        bufs = (tok0_v, tok1_v)
        n = len(_SCHED)

        pltpu.sync_copy(xw_hbm.at[wid], idx_v)
        pos_cp = pltpu.async_copy(pos_hbm.at[pl.ds(p0, _HALF)], pos_v, psem)

        def gather(ci):
            _, b, k = _SCHED[ci]
            return pltpu.async_copy(tok_hbm.at[idx_v.at[b * _KPB + k]],
                                    bufs[ci % 2], gsem)

        def store(ci):
            _, b, k = _SCHED[ci]
            rows = pl.multiple_of(b * _T + p0 + k * _CK, _CK)
            return pltpu.async_copy(bufs[ci % 2],
                                    out_hbm.at[pl.ds(rows, _CK)], ssem)

        g_cp = {0: gather(0)}
        s_cp = {}
        for ci, (h, b, k) in enumerate(_SCHED):
            buf = bufs[ci % 2]
            g_cp.pop(ci).wait()
            if ci - 1 in s_cp:
                s_cp.pop(ci - 1).wait()
            if ci + 1 < n:
                g_cp[ci + 1] = gather(ci + 1)
            if ci == 0 or ci == n // 2:
                pos_cp.wait()
            pb = (k - 2 * h) * _CK

            def row_body(i, _, pb=pb, buf=buf):
                for g in range(_GRP):
                    s = pl.ds(g * _LANES, _LANES)
                    buf[i, s] = buf[i, s] + pos_v[pb + i, s]
                return 0

            lax.fori_loop(0, _CK, row_body, 0)
            if ci == n // 2 - 1:
                pos_cp = pltpu.async_copy(
                    pos_hbm.at[pl.ds(p0 + _HALF, _HALF)], pos_v, psem)
            s_cp[ci] = store(ci)
        s_cp.pop(n - 1).wait()

    return emb


_emb = _make_emb_kernel()


def kernel(x, token_table, pos_table):
    xw = (x.reshape(_B, _NW, _P_W)
           .transpose(1, 0, 2)
           .reshape(_NW, _B * _KPB, _CK)
           .astype(jnp.int32))
    out = _emb(xw, token_table, pos_table)
    return out.reshape(_B, _T, _D)


# speedup vs baseline: 1.6232x; 1.2542x over previous
import functools

import jax
import jax.numpy as jnp
from jax import lax
from jax.experimental import pallas as pl
from jax.experimental.pallas import tpu as pltpu
from jax.experimental.pallas import tpu_sc as plsc

_B = 4
_T = 4096
_D = 768
_N = _B * _T
_NC = 2
_NS = 16
_NW = _NC * _NS
_P_W = _T // _NW
_CK = 32
_KPB = _P_W // _CK
_HALF = _P_W // 2
_LANES = 16
_GRP = _D // _LANES

_SCHED = [(h, b, k) for h in (0, 1) for b in range(_B)
          for k in (2 * h, 2 * h + 1)]


def _make_emb_kernel():
    mesh = plsc.VectorSubcoreMesh(core_axis_name="c", subcore_axis_name="s")

    @functools.partial(
        pl.kernel,
        out_type=jax.ShapeDtypeStruct((_N, _D), jnp.float32),
        mesh=mesh,
        scratch_types=[
            pltpu.VMEM((_B * _KPB, _CK), jnp.int32),
            pltpu.VMEM((_HALF, _D), jnp.float32),
            pltpu.VMEM((_CK, _D), jnp.float32),
            pltpu.VMEM((_CK, _D), jnp.float32),
            pltpu.VMEM((_CK, _D), jnp.float32),
            pltpu.SemaphoreType.DMA,
            pltpu.SemaphoreType.DMA,
            pltpu.SemaphoreType.DMA,
        ],
    )
    def emb(xw_hbm, tok_hbm, pos_hbm, out_hbm,
            idx_v, pos_v, tok0_v, tok1_v, tok2_v, psem, gsem, ssem):
        wid = lax.axis_index("s") * _NC + lax.axis_index("c")
        p0 = pl.multiple_of(wid * _P_W, _P_W)
        bufs = (tok0_v, tok1_v, tok2_v)
        n = len(_SCHED)

        pltpu.sync_copy(xw_hbm.at[wid], idx_v)
        pos_cp = pltpu.async_copy(pos_hbm.at[pl.ds(p0, _HALF)], pos_v, psem)

        def gather(ci):
            _, b, k = _SCHED[ci]
            return pltpu.async_copy(tok_hbm.at[idx_v.at[b * _KPB + k]],
                                    bufs[ci % 3], gsem)

        def store(ci):
            _, b, k = _SCHED[ci]
            rows = pl.multiple_of(b * _T + p0 + k * _CK, _CK)
            return pltpu.async_copy(bufs[ci % 3],
                                    out_hbm.at[pl.ds(rows, _CK)], ssem)

        g_cp = {0: gather(0)}
        s_cp = {}
        for ci, (h, b, k) in enumerate(_SCHED):
            buf = bufs[ci % 3]
            g_cp.pop(ci).wait()
            if ci - 2 in s_cp:
                s_cp.pop(ci - 2).wait()
            if ci + 1 < n:
                g_cp[ci + 1] = gather(ci + 1)
            if ci == 0 or ci == n // 2:
                pos_cp.wait()
            pb = (k - 2 * h) * _CK

            def row_body(i, _, pb=pb, buf=buf):
                for g in range(_GRP):
                    s = pl.ds(g * _LANES, _LANES)
                    buf[i, s] = buf[i, s] + pos_v[pb + i, s]
                return 0

            lax.fori_loop(0, _CK, row_body, 0)
            if ci == n // 2 - 1:
                pos_cp = pltpu.async_copy(
                    pos_hbm.at[pl.ds(p0 + _HALF, _HALF)], pos_v, psem)
            s_cp[ci] = store(ci)
        s_cp.pop(n - 2).wait()
        s_cp.pop(n - 1).wait()

    return emb


_emb = _make_emb_kernel()


def kernel(x, token_table, pos_table):
    xw = (x.reshape(_B, _NW, _P_W)
           .transpose(1, 0, 2)
           .reshape(_NW, _B * _KPB, _CK)
           .astype(jnp.int32))
    out = _emb(xw, token_table, pos_table)
    return out.reshape(_B, _T, _D)


# speedup vs baseline: 2.0458x; 1.2604x over previous
import functools

import jax
import jax.numpy as jnp
from jax import lax
from jax.experimental import pallas as pl
from jax.experimental.pallas import tpu as pltpu
from jax.experimental.pallas import tpu_sc as plsc

_B = 4
_T = 4096
_D = 768
_N = _B * _T
_NC = 2
_NS = 16
_NW = _NC * _NS
_P_W = _T // _NW
_CK = 32
_KPB = _P_W // _CK
_HALF = _P_W // 2
_LANES = 16
_GRP = _D // _LANES

_SCHED = [(h, b, k) for h in (0, 1) for b in range(_B)
          for k in (2 * h, 2 * h + 1)]


def _make_emb_kernel():
    mesh = plsc.VectorSubcoreMesh(core_axis_name="c", subcore_axis_name="s")

    @functools.partial(
        pl.kernel,
        out_type=jax.ShapeDtypeStruct((_N, _D), jnp.float32),
        mesh=mesh,
        scratch_types=[
            pltpu.VMEM((_B * _KPB, _CK), jnp.int32),
            pltpu.VMEM((_HALF, _D), jnp.float32),
            pltpu.VMEM((_CK, _D), jnp.float32),
            pltpu.VMEM((_CK, _D), jnp.float32),
            pltpu.VMEM((_CK, _D), jnp.float32),
            pltpu.SemaphoreType.DMA,
            pltpu.SemaphoreType.DMA,
            pltpu.SemaphoreType.DMA,
        ],
    )
    def emb(xw_hbm, tok_hbm, pos_hbm, out_hbm,
            idx_v, pos_v, tok0_v, tok1_v, tok2_v, psem, gsem, ssem):
        wid = lax.axis_index("s") * _NC + lax.axis_index("c")
        p0 = pl.multiple_of(wid * _P_W, _P_W)
        bufs = (tok0_v, tok1_v, tok2_v)
        n = len(_SCHED)

        pltpu.sync_copy(xw_hbm.at[wid], idx_v)
        pos_cp = pltpu.async_copy(pos_hbm.at[pl.ds(p0, _HALF)], pos_v, psem)

        def gather(ci):
            _, b, k = _SCHED[ci]
            return pltpu.async_copy(tok_hbm.at[idx_v.at[b * _KPB + k]],
                                    bufs[ci % 3], gsem)

        def store(ci):
            _, b, k = _SCHED[ci]
            rows = pl.multiple_of(b * _T + p0 + k * _CK, _CK)
            return pltpu.async_copy(bufs[ci % 3],
                                    out_hbm.at[pl.ds(rows, _CK)], ssem)

        g_cp = {0: gather(0)}
        s_cp = {}
        for ci, (h, b, k) in enumerate(_SCHED):
            buf = bufs[ci % 3]
            g_cp.pop(ci).wait()
            if ci - 2 in s_cp:
                s_cp.pop(ci - 2).wait()
            if ci + 1 < n:
                g_cp[ci + 1] = gather(ci + 1)
            if ci == 0 or ci == n // 2:
                pos_cp.wait()
            pb = (k - 2 * h) * _CK

            @plsc.parallel_loop(0, _CK, unroll=1)
            def _(i, pb=pb, buf=buf):
                for g in range(_GRP):
                    s = pl.ds(g * _LANES, _LANES)
                    buf[i, s] = buf[i, s] + pos_v[pb + i, s]
            if ci == n // 2 - 1:
                pos_cp = pltpu.async_copy(
                    pos_hbm.at[pl.ds(p0 + _HALF, _HALF)], pos_v, psem)
            s_cp[ci] = store(ci)
        s_cp.pop(n - 2).wait()
        s_cp.pop(n - 1).wait()

    return emb


_emb = _make_emb_kernel()


def kernel(x, token_table, pos_table):
    xw = (x.reshape(_B, _NW, _P_W)
           .transpose(1, 0, 2)
           .reshape(_NW, _B * _KPB, _CK)
           .astype(jnp.int32))
    out = _emb(xw, token_table, pos_table)
    return out.reshape(_B, _T, _D)


# speedup vs baseline: 2.0819x; 1.0176x over previous
import functools

import jax
import jax.numpy as jnp
from jax import lax
from jax.experimental import pallas as pl
from jax.experimental.pallas import tpu as pltpu
from jax.experimental.pallas import tpu_sc as plsc

_B = 4
_T = 4096
_D = 768
_N = _B * _T
_NC = 2
_NS = 16
_NW = _NC * _NS
_P_W = _T // _NW
_CK = 32
_KPB = _P_W // _CK
_HALF = _P_W // 2
_LANES = 16
_GRP = _D // _LANES

_SCHED = [(h, b, k) for h in (0, 1) for b in range(_B)
          for k in (2 * h, 2 * h + 1)]


def _make_emb_kernel():
    mesh = plsc.VectorSubcoreMesh(core_axis_name="c", subcore_axis_name="s")

    @functools.partial(
        pl.kernel,
        out_type=jax.ShapeDtypeStruct((_N, _D), jnp.float32),
        mesh=mesh,
        scratch_types=[
            pltpu.VMEM((_B * _KPB, _CK), jnp.int32),
            pltpu.VMEM((_HALF, _D), jnp.float32),
            pltpu.VMEM((_CK, _D), jnp.float32),
            pltpu.VMEM((_CK, _D), jnp.float32),
            pltpu.VMEM((_CK, _D), jnp.float32),
            pltpu.SemaphoreType.DMA,
            pltpu.SemaphoreType.DMA,
            pltpu.SemaphoreType.DMA,
        ],
    )
    def emb(xw_hbm, tok_hbm, pos_hbm, out_hbm,
            idx_v, pos_v, tok0_v, tok1_v, tok2_v, psem, gsem, ssem):
        wid = lax.axis_index("s") * _NC + lax.axis_index("c")
        p0 = pl.multiple_of(wid * _P_W, _P_W)
        bufs = (tok0_v, tok1_v, tok2_v)
        n = len(_SCHED)

        pltpu.sync_copy(xw_hbm.at[wid], idx_v)
        pos_cp = pltpu.async_copy(pos_hbm.at[pl.ds(p0, _HALF)], pos_v, psem)

        def gather(ci):
            _, b, k = _SCHED[ci]
            return pltpu.async_copy(tok_hbm.at[idx_v.at[b * _KPB + k]],
                                    bufs[ci % 3], gsem)

        def store(ci):
            _, b, k = _SCHED[ci]
            rows = pl.multiple_of(b * _T + p0 + k * _CK, _CK)
            return pltpu.async_copy(bufs[ci % 3],
                                    out_hbm.at[pl.ds(rows, _CK)], ssem)

        g_cp = {0: gather(0)}
        s_cp = {}
        for ci, (h, b, k) in enumerate(_SCHED):
            buf = bufs[ci % 3]
            g_cp.pop(ci).wait()
            if ci - 2 in s_cp:
                s_cp.pop(ci - 2).wait()
            if ci + 1 < n:
                g_cp[ci + 1] = gather(ci + 1)
            if ci == 0 or ci == n // 2:
                pos_cp.wait()
            pb = (k - 2 * h) * _CK

            @plsc.parallel_loop(0, _CK, unroll=1)
            def _(i, pb=pb, buf=buf):
                for g in range(_GRP):
                    s = pl.ds(g * _LANES, _LANES)
                    plsc.addupdate(buf.at[i, s], pos_v[pb + i, s])
            if ci == n // 2 - 1:
                pos_cp = pltpu.async_copy(
                    pos_hbm.at[pl.ds(p0 + _HALF, _HALF)], pos_v, psem)
            s_cp[ci] = store(ci)
        s_cp.pop(n - 2).wait()
        s_cp.pop(n - 1).wait()

    return emb


_emb = _make_emb_kernel()


def kernel(x, token_table, pos_table):
    xw = (x.reshape(_B, _NW, _P_W)
           .transpose(1, 0, 2)
           .reshape(_NW, _B * _KPB, _CK)
           .astype(jnp.int32))
    out = _emb(xw, token_table, pos_table)
    return out.reshape(_B, _T, _D)


# speedup vs baseline: 2.1511x; 1.0332x over previous
import functools

import jax
import jax.numpy as jnp
from jax import lax
from jax.experimental import pallas as pl
from jax.experimental.pallas import tpu as pltpu
from jax.experimental.pallas import tpu_sc as plsc

_B = 4
_T = 4096
_D = 768
_N = _B * _T
_NC = 2
_NS = 16
_NW = _NC * _NS
_P_W = _T // _NW
_CK = 32
_KPB = _P_W // _CK
_LANES = 16
_GRP = _D // _LANES

_SCHED = [(q, b) for q in range(_KPB) for b in range(_B)]


def _make_emb_kernel():
    mesh = plsc.VectorSubcoreMesh(core_axis_name="c", subcore_axis_name="s")

    @functools.partial(
        pl.kernel,
        out_type=jax.ShapeDtypeStruct((_N, _D), jnp.float32),
        mesh=mesh,
        scratch_types=[
            pltpu.VMEM((_B * _KPB, _CK), jnp.int32),
            pltpu.VMEM((_CK, _D), jnp.float32),
            pltpu.VMEM((_CK, _D), jnp.float32),
            pltpu.VMEM((_CK, _D), jnp.float32),
            pltpu.VMEM((_CK, _D), jnp.float32),
            pltpu.VMEM((_CK, _D), jnp.float32),
            pltpu.SemaphoreType.DMA,
            pltpu.SemaphoreType.DMA,
            pltpu.SemaphoreType.DMA,
        ],
    )
    def emb(xw_hbm, tok_hbm, pos_hbm, out_hbm,
            idx_v, posa_v, posb_v, tok0_v, tok1_v, tok2_v, psem, gsem, ssem):
        wid = lax.axis_index("s") * _NC + lax.axis_index("c")
        p0 = pl.multiple_of(wid * _P_W, _P_W)
        bufs = (tok0_v, tok1_v, tok2_v)
        pos_bufs = (posa_v, posb_v)
        n = len(_SCHED)

        pltpu.sync_copy(xw_hbm.at[wid], idx_v)

        def load_pos(q):
            rows = pl.multiple_of(p0 + q * _CK, _CK)
            return pltpu.async_copy(pos_hbm.at[pl.ds(rows, _CK)],
                                    pos_bufs[q % 2], psem)

        def gather(ci):
            q, b = _SCHED[ci]
            return pltpu.async_copy(tok_hbm.at[idx_v.at[b * _KPB + q]],
                                    bufs[ci % 3], gsem)

        def store(ci):
            q, b = _SCHED[ci]
            rows = pl.multiple_of(b * _T + p0 + q * _CK, _CK)
            return pltpu.async_copy(bufs[ci % 3],
                                    out_hbm.at[pl.ds(rows, _CK)], ssem)

        pos_cp = load_pos(0)
        g_cp = {0: gather(0)}
        s_cp = {}
        for ci, (q, b) in enumerate(_SCHED):
            buf = bufs[ci % 3]
            pos_v = pos_bufs[q % 2]
            g_cp.pop(ci).wait()
            if ci - 2 in s_cp:
                s_cp.pop(ci - 2).wait()
            if ci + 1 < n:
                g_cp[ci + 1] = gather(ci + 1)
            if b == 0:
                pos_cp.wait()
                if q + 1 < _KPB:
                    pos_cp = load_pos(q + 1)

            @plsc.parallel_loop(0, _CK, unroll=1)
            def _(i, buf=buf, pos_v=pos_v):
                for g in range(_GRP):
                    s = pl.ds(g * _LANES, _LANES)
                    plsc.addupdate(buf.at[i, s], pos_v[i, s])

            s_cp[ci] = store(ci)
        s_cp.pop(n - 2).wait()
        s_cp.pop(n - 1).wait()

    return emb


_emb = _make_emb_kernel()


def kernel(x, token_table, pos_table):
    xw = (x.reshape(_B, _NW, _P_W)
           .transpose(1, 0, 2)
           .reshape(_NW, _B * _KPB, _CK)
           .astype(jnp.int32))
    out = _emb(xw, token_table, pos_table)
    return out.reshape(_B, _T, _D)
